# baseline (speedup 1.0000x reference)
import functools

import numpy as np
import jax
import jax.numpy as jnp
from jax import lax
from jax.experimental import pallas as pl
from jax.experimental.pallas import tpu as pltpu
from jax.experimental.pallas import tpu_sc as plsc

_NS = 500
_NR = 49
_CC = 48
_SPT = _NS // 4
_FMAX = np.float32(3.0e38)


def _pool_matrix():
    m = np.zeros((196, _NR), np.float32)
    for r in range(_NR):
        ri, rj = r // 7, r % 7
        for ki in range(2):
            for kj in range(2):
                m[(2 * ri + ki) * 14 + (2 * rj + kj), r] = 1.0
    return m


def _sc_indicator_body(score_hbm, sig_hbm, noise_hbm, gid_hbm, w_hbm,
                       score_v, noise_v, row_v, tmp_v, sig_v, gid_v, ind_v,
                       hist_sh, ind_sh):
    core = lax.axis_index("c")
    tid = lax.axis_index("s")
    lr = tid // 4
    q = tid % 4
    g = core * 4 + lr
    pltpu.sync_copy(score_hbm.at[pl.ds(g * 256, 256)], score_v)
    pltpu.sync_copy(sig_hbm, sig_v)
    pltpu.sync_copy(gid_hbm.at[pl.ds((core * 4 + tid) * 16, 16)], gid_v)
    pltpu.sync_copy(noise_hbm.at[pl.ds(g * 32768, 32768)], noise_v)

    def fsp(x):
        return jnp.full((16,), x, jnp.float32)

    def isp(x):
        return jnp.full((16,), x, jnp.int32)

    sig = sig_v[...]
    it = lax.iota(jnp.int32, 16)
    one = fsp(np.float32(1.0))
    zero = fsp(np.float32(0.0))
    itmask = [it == isp(l) for l in range(16)]

    tv = [[score_v[pl.ds(j * 64 + 16 * kc, 16)] for kc in range(4)]
          for j in range(4)]
    pool_s = []
    mn = np.float32(_FMAX)
    mx = np.float32(-_FMAX)
    for r in range(_NR):
        kc, l = r // 16, r % 16
        p = (((tv[0][kc][l] + tv[1][kc][l]) + tv[2][kc][l])
             + tv[3][kc][l]) * np.float32(0.25)
        pool_s.append(p)
        mn = jnp.minimum(mn, p)
        mx = jnp.maximum(mx, p)
    inv_v = one / (fsp(mx - mn) + fsp(np.float32(1e-5)))

    qo = q * 128
    m_v = [fsp(np.float32(-_FMAX)) for _ in range(8)]
    b_v = [isp(63) for _ in range(8)]
    for r in range(_NR):
        snsr = fsp(pool_s[r] - mn) * inv_v
        for v in range(8):
            pert = snsr + noise_v[pl.ds(r * 512 + qo + v * 16, 16)] * sig
            upd = pert > m_v[v]
            m_v[v] = jnp.where(upd, pert, m_v[v])
            b_v[v] = jnp.where(upd, isp(r), b_v[v])
    for v in range(8):
        valid = (isp(q * 128 + v * 16) + it) < isp(_NS)
        b_v[v] = jnp.where(valid, b_v[v], isp(63))

    hist = [zero, zero, zero, zero]
    for r in range(_NR):
        cntv = zero
        for v in range(8):
            cntv = cntv + jnp.where(b_v[v] == isp(r), one, zero)
        s = cntv[0]
        for i in range(1, 16):
            s = s + cntv[i]
        kc, l = r // 16, r % 16
        hist[kc] = hist[kc] + jnp.where(itmask[l], fsp(s), zero)
    for kc in range(4):
        row_v[pl.ds(kc * 16, 16)] = hist[kc]
    pltpu.sync_copy(row_v, hist_sh.at[pl.ds(tid * 64, 64)])
    plsc.subcore_barrier()

    @pl.when(tid < 4)
    def _sum_row():
        acc = [jnp.zeros((16,), jnp.float32) for _ in range(4)]
        for j in range(4):
            pltpu.sync_copy(hist_sh.at[pl.ds((tid * 4 + j) * 64, 64)],
                            tmp_v)
            for kc in range(4):
                acc[kc] = acc[kc] + tmp_v[pl.ds(kc * 16, 16)]
        ns = jnp.full((16,), np.float32(_NS), jnp.float32)
        for kc in range(4):
            row_v[pl.ds(kc * 16, 16)] = acc[kc] / ns
        pltpu.sync_copy(row_v, ind_sh.at[pl.ds(tid * 64, 64)])

    plsc.subcore_barrier()

    @pl.when(tid < 4)
    def _route():
        bt = core * 4 + tid
        pltpu.sync_copy(ind_sh, ind_v)
        key = gid_v[...]
        for kc in range(4):
            wk = zero
            for j in range(4):
                wk = wk + jnp.where(key == isp(j),
                                    ind_v[pl.ds(j * 64 + kc * 16, 16)],
                                    zero)
            row_v[pl.ds(kc * 16, 16)] = wk
        pltpu.sync_copy(row_v, w_hbm.at[pl.ds(bt * 64, 64)])


def _mix_body(w_ref, x_ref, o_ref):
    xb = x_ref[0, :, 0]
    wp = lax.broadcasted_iota(jnp.int32, (224, 64), 0)
    q = lax.broadcasted_iota(jnp.int32, (224, 64), 1)
    rj = wp // 32
    kj = (wp % 32) // 16
    aj = wp % 16
    ajq = q // 4
    kiq = (q // 2) % 2
    kjq = q % 2
    colmask = (ajq == aj) & (kjq == kj)
    kimask = [colmask & (kiq == 0), colmask & (kiq == 1)]
    acc = jnp.zeros((_CC * 16, 64), jnp.float32)
    for ri in range(7):
        wsel = jnp.zeros((224, 64), jnp.float32)
        for j in range(7):
            wsel = wsel + jnp.where(rj == j, w_ref[0, 0, 0, ri * 7 + j], 0.0)
        for ki in range(2):
            b = jnp.where(kimask[ki], wsel, 0.0)
            base = ri * 32 + ki * 16
            xs = xb[:, base:base + 16, :].reshape(_CC * 16, 224)
            acc = acc + lax.dot_general(
                xs, b, (((1,), (0,)), ((), ())),
                preferred_element_type=jnp.float32)
    o_ref[0, 0] = acc.reshape(_CC, 16, 64)


def _compute_w(score, sigma, group_id, noise):
    score2 = score.reshape(8, 14, 14)
    score_pre = jnp.stack(
        [score2[:, i::2, j::2].reshape(8, _NR)
         for (i, j) in ((0, 0), (0, 1), (1, 0), (1, 1))], axis=1)
    score_pre = jnp.pad(score_pre,
                        ((0, 0), (0, 0), (0, 64 - _NR))).reshape(-1)
    sig16 = jnp.full((16,), sigma, jnp.float32)
    noise_t = jnp.transpose(jnp.pad(noise, ((0, 0), (0, 12), (0, 15))),
                            (0, 2, 1)).reshape(-1)
    gid_splat = jnp.broadcast_to(
        group_id.reshape(8, 1).astype(jnp.int32), (8, 16))
    gid_splat = jnp.pad(gid_splat, ((0, 8), (0, 0))).reshape(-1)
    mesh = plsc.VectorSubcoreMesh(core_axis_name="c", subcore_axis_name="s")
    k = functools.partial(
        pl.kernel,
        out_type=jax.ShapeDtypeStruct((512,), jnp.float32),
        mesh=mesh,
        scratch_types=[
            pltpu.VMEM((256,), jnp.float32),
            pltpu.VMEM((32768,), jnp.float32),
            pltpu.VMEM((64,), jnp.float32),
            pltpu.VMEM((64,), jnp.float32),
            pltpu.VMEM((16,), jnp.float32),
            pltpu.VMEM((16,), jnp.int32),
            pltpu.VMEM((256,), jnp.float32),
            pltpu.VMEM_SHARED((1024,), jnp.float32),
            pltpu.VMEM_SHARED((256,), jnp.float32),
        ],
    )(_sc_indicator_body)
    w_p = k(score_pre, sig16, noise_t, gid_splat).reshape(8, 64)
    return w_p[:, :_NR].reshape(2, 4, 1, _NR)


def _mix(x, w, interpret=False):
    b, c, t = 2, 96, 4
    out = pl.pallas_call(
        _mix_body,
        grid=(b, t, c // _CC),
        in_specs=[
            pl.BlockSpec((1, 1, 1, _NR), lambda bi, ti, ci: (bi, ti, 0, 0)),
            pl.BlockSpec((1, _CC, 1, 224, 224),
                         lambda bi, ti, ci: (bi, ci, ti, 0, 0)),
        ],
        out_specs=pl.BlockSpec((1, 1, _CC, 16, 64),
                               lambda bi, ti, ci: (bi, ti, ci, 0, 0)),
        out_shape=jax.ShapeDtypeStruct((b, t, c, 16, 64), jnp.float32),
        interpret=interpret,
    )(w, x)
    return out.reshape(b * t, 1, c * 1024)


def kernel(x, score, sigma, group_id, noise):
    w = _compute_w(score, sigma, group_id, noise)
    return _mix(x, w)


# speedup vs baseline: 1.0180x; 1.0180x over previous
import functools

import numpy as np
import jax
import jax.numpy as jnp
from jax import lax
from jax.experimental import pallas as pl
from jax.experimental.pallas import tpu as pltpu
from jax.experimental.pallas import tpu_sc as plsc

_NS = 500
_NR = 49
_CC = 48
_SPT = _NS // 4
_FMAX = np.float32(3.0e38)


def _pool_matrix():
    m = np.zeros((196, _NR), np.float32)
    for r in range(_NR):
        ri, rj = r // 7, r % 7
        for ki in range(2):
            for kj in range(2):
                m[(2 * ri + ki) * 14 + (2 * rj + kj), r] = 1.0
    return m


def _sc_indicator_body(score_hbm, sig_hbm, noise_hbm, gid_hbm, w_hbm,
                       score_v, noise_v, row_v, tmp_v, sig_v, gid_v, ind_v,
                       hist_sh, ind_sh):
    core = lax.axis_index("c")
    tid = lax.axis_index("s")
    lr = tid // 4
    q = tid % 4
    g = core * 4 + lr
    pltpu.sync_copy(score_hbm.at[pl.ds(g * 256, 256)], score_v)
    pltpu.sync_copy(sig_hbm, sig_v)
    pltpu.sync_copy(gid_hbm.at[pl.ds((core * 4 + tid) * 16, 16)], gid_v)
    pltpu.sync_copy(noise_hbm.at[pl.ds(g * 32768, 32768)], noise_v)

    def fsp(x):
        return jnp.full((16,), x, jnp.float32)

    def isp(x):
        return jnp.full((16,), x, jnp.int32)

    sig = sig_v[...]
    it = lax.iota(jnp.int32, 16)
    one = fsp(np.float32(1.0))
    zero = fsp(np.float32(0.0))
    itmask = [it == isp(l) for l in range(16)]

    tv = [[score_v[pl.ds(j * 64 + 16 * kc, 16)] for kc in range(4)]
          for j in range(4)]
    pool_s = []
    mn = np.float32(_FMAX)
    mx = np.float32(-_FMAX)
    for r in range(_NR):
        kc, l = r // 16, r % 16
        p = (((tv[0][kc][l] + tv[1][kc][l]) + tv[2][kc][l])
             + tv[3][kc][l]) * np.float32(0.25)
        pool_s.append(p)
        mn = jnp.minimum(mn, p)
        mx = jnp.maximum(mx, p)
    inv_v = one / (fsp(mx - mn) + fsp(np.float32(1e-5)))

    qo = q * 128
    m_v = [fsp(np.float32(-_FMAX)) for _ in range(8)]
    b_v = [isp(63) for _ in range(8)]
    for r in range(_NR):
        snsr = fsp(pool_s[r] - mn) * inv_v
        for v in range(8):
            pert = snsr + noise_v[pl.ds(r * 512 + qo + v * 16, 16)] * sig
            upd = pert > m_v[v]
            m_v[v] = jnp.where(upd, pert, m_v[v])
            b_v[v] = jnp.where(upd, isp(r), b_v[v])
    for v in range(8):
        valid = (isp(q * 128 + v * 16) + it) < isp(_NS)
        b_v[v] = jnp.where(valid, b_v[v], isp(63))

    tmp_v[pl.ds(16, 16)] = zero
    hist = [zero, zero, zero, zero]
    for r in range(_NR):
        cntv = zero
        for v in range(8):
            cntv = cntv + jnp.where(b_v[v] == isp(r), one, zero)
        for sh in (8, 4, 2):
            tmp_v[pl.ds(0, 16)] = cntv
            cntv = cntv + tmp_v[pl.ds(sh, 16)]
        s = cntv[0] + cntv[1]
        kc, l = r // 16, r % 16
        hist[kc] = hist[kc] + jnp.where(itmask[l], fsp(s), zero)
    for kc in range(4):
        row_v[pl.ds(kc * 16, 16)] = hist[kc]
    pltpu.sync_copy(row_v, hist_sh.at[pl.ds(tid * 64, 64)])
    plsc.subcore_barrier()

    @pl.when(tid < 4)
    def _sum_row():
        acc = [jnp.zeros((16,), jnp.float32) for _ in range(4)]
        for j in range(4):
            pltpu.sync_copy(hist_sh.at[pl.ds((tid * 4 + j) * 64, 64)],
                            tmp_v)
            for kc in range(4):
                acc[kc] = acc[kc] + tmp_v[pl.ds(kc * 16, 16)]
        ns = jnp.full((16,), np.float32(_NS), jnp.float32)
        for kc in range(4):
            row_v[pl.ds(kc * 16, 16)] = acc[kc] / ns
        pltpu.sync_copy(row_v, ind_sh.at[pl.ds(tid * 64, 64)])

    plsc.subcore_barrier()

    @pl.when(tid < 4)
    def _route():
        bt = core * 4 + tid
        pltpu.sync_copy(ind_sh, ind_v)
        key = gid_v[...]
        for kc in range(4):
            wk = zero
            for j in range(4):
                wk = wk + jnp.where(key == isp(j),
                                    ind_v[pl.ds(j * 64 + kc * 16, 16)],
                                    zero)
            row_v[pl.ds(kc * 16, 16)] = wk
        pltpu.sync_copy(row_v, w_hbm.at[pl.ds(bt * 64, 64)])


def _mix_body(w_ref, x_ref, o_ref):
    xb = x_ref[0, :, 0]
    wp = lax.broadcasted_iota(jnp.int32, (224, 64), 0)
    q = lax.broadcasted_iota(jnp.int32, (224, 64), 1)
    rj = wp // 32
    kj = (wp % 32) // 16
    aj = wp % 16
    ajq = q // 4
    kiq = (q // 2) % 2
    kjq = q % 2
    colmask = (ajq == aj) & (kjq == kj)
    kimask = [colmask & (kiq == 0), colmask & (kiq == 1)]
    acc = jnp.zeros((_CC * 16, 64), jnp.float32)
    for ri in range(7):
        wsel = jnp.zeros((224, 64), jnp.float32)
        for j in range(7):
            wsel = wsel + jnp.where(rj == j, w_ref[0, 0, ri * 7 + j], 0.0)
        for ki in range(2):
            b = jnp.where(kimask[ki], wsel, 0.0)
            base = ri * 32 + ki * 16
            xs = xb[:, base:base + 16, :].reshape(_CC * 16, 224)
            acc = acc + lax.dot_general(
                xs, b, (((1,), (0,)), ((), ())),
                preferred_element_type=jnp.float32)
    o_ref[0, 0] = acc.reshape(_CC, 16, 64)


def _compute_w(score, sigma, group_id, noise):
    score2 = score.reshape(8, 14, 14)
    score_pre = jnp.stack(
        [score2[:, i::2, j::2].reshape(8, _NR)
         for (i, j) in ((0, 0), (0, 1), (1, 0), (1, 1))], axis=1)
    score_pre = jnp.pad(score_pre,
                        ((0, 0), (0, 0), (0, 64 - _NR))).reshape(-1)
    sig16 = jnp.full((16,), sigma, jnp.float32)
    noise_t = jnp.transpose(jnp.pad(noise, ((0, 0), (0, 12), (0, 15))),
                            (0, 2, 1)).reshape(-1)
    gid_splat = jnp.broadcast_to(
        group_id.reshape(8, 1).astype(jnp.int32), (8, 16))
    gid_splat = jnp.pad(gid_splat, ((0, 8), (0, 0))).reshape(-1)
    mesh = plsc.VectorSubcoreMesh(core_axis_name="c", subcore_axis_name="s")
    k = functools.partial(
        pl.kernel,
        out_type=jax.ShapeDtypeStruct((512,), jnp.float32),
        mesh=mesh,
        scratch_types=[
            pltpu.VMEM((256,), jnp.float32),
            pltpu.VMEM((32768,), jnp.float32),
            pltpu.VMEM((64,), jnp.float32),
            pltpu.VMEM((64,), jnp.float32),
            pltpu.VMEM((16,), jnp.float32),
            pltpu.VMEM((16,), jnp.int32),
            pltpu.VMEM((256,), jnp.float32),
            pltpu.VMEM_SHARED((1024,), jnp.float32),
            pltpu.VMEM_SHARED((256,), jnp.float32),
        ],
    )(_sc_indicator_body)
    return k(score_pre, sig16, noise_t, gid_splat).reshape(8, 1, 64)


def _mix(x, w, interpret=False):
    b, c, t = 2, 96, 4
    out = pl.pallas_call(
        _mix_body,
        grid=(b, t, c // _CC),
        in_specs=[
            pl.BlockSpec((1, 1, 64), lambda bi, ti, ci: (bi * 4 + ti, 0, 0)),
            pl.BlockSpec((1, _CC, 1, 224, 224),
                         lambda bi, ti, ci: (bi, ci, ti, 0, 0)),
        ],
        out_specs=pl.BlockSpec((1, 1, _CC, 16, 64),
                               lambda bi, ti, ci: (bi, ti, ci, 0, 0)),
        out_shape=jax.ShapeDtypeStruct((b, t, c, 16, 64), jnp.float32),
        interpret=interpret,
    )(w, x)
    return out.reshape(b * t, 1, c * 1024)


def kernel(x, score, sigma, group_id, noise):
    w = _compute_w(score, sigma, group_id, noise)
    return _mix(x, w)
